# Initial kernel scaffold; baseline (speedup 1.0000x reference)
import jax
import jax.numpy as jnp
from jax.experimental import pallas as pl


def kernel(node_features, edge_features, edge_src, edge_dst, params):
    raise NotImplementedError("write your pallas kernel here")


# speedup vs baseline: 15.0334x; 15.0334x over previous
import functools

import jax
import jax.numpy as jnp
from jax import lax
from jax.experimental import pallas as pl
from jax.experimental.pallas import tpu as pltpu
from jax.experimental.pallas import tpu_sc as plsc

F32 = jnp.float32
NC, NS = 2, 16
NW = NC * NS
LANE = 16


def _sc_mesh():
    return plsc.VectorSubcoreMesh(core_axis_name="c", subcore_axis_name="s",
                                  num_cores=NC, num_subcores=NS)


def _proj(x, ws, wd):
    BN, D = x.shape
    H = ws.shape[1]
    T = 2048
    def body(x_ref, ws_ref, wd_ref, p_ref, q_ref):
        xv = x_ref[...]
        p_ref[...] = jnp.dot(xv, ws_ref[...], preferred_element_type=F32)
        q_ref[...] = jnp.dot(xv, wd_ref[...], preferred_element_type=F32)
    return pl.pallas_call(
        body,
        grid=(BN // T,),
        in_specs=[pl.BlockSpec((T, D), lambda i: (i, 0)),
                  pl.BlockSpec((D, H), lambda i: (0, 0)),
                  pl.BlockSpec((D, H), lambda i: (0, 0))],
        out_specs=[pl.BlockSpec((T, H), lambda i: (i, 0)),
                   pl.BlockSpec((T, H), lambda i: (i, 0))],
        out_shape=[jax.ShapeDtypeStruct((BN, H), F32),
                   jax.ShapeDtypeStruct((BN, H), F32)],
    )(x, ws, wd)


def _edge_mlp(g, ef, we, b1, w2, b2):
    BE, H = g.shape
    DE = ef.shape[1]
    T = 4096
    def body(g_ref, ef_ref, we_ref, b1_ref, w2_ref, b2_ref, m_ref):
        h = (g_ref[...]
             + jnp.dot(ef_ref[...], we_ref[...], preferred_element_type=F32)
             + b1_ref[...])
        h = jnp.maximum(h, 0.0)
        m = jnp.dot(h, w2_ref[...], preferred_element_type=F32) + b2_ref[...]
        m_ref[...] = jnp.maximum(m, 0.0)
    return pl.pallas_call(
        body,
        grid=(BE // T,),
        in_specs=[pl.BlockSpec((T, H), lambda i: (i, 0)),
                  pl.BlockSpec((T, DE), lambda i: (i, 0)),
                  pl.BlockSpec((DE, H), lambda i: (0, 0)),
                  pl.BlockSpec((1, H), lambda i: (0, 0)),
                  pl.BlockSpec((H, H), lambda i: (0, 0)),
                  pl.BlockSpec((1, H), lambda i: (0, 0))],
        out_specs=pl.BlockSpec((T, H), lambda i: (i, 0)),
        out_shape=jax.ShapeDtypeStruct((BE, H), F32),
    )(g, ef, we, b1, w2, b2)


def _node_mlp(x, agg, wua, wub, bu1, wu2, bu2, residual):
    BN, H = x.shape
    T = 2048
    with_res = residual is not None
    def body(*refs):
        if with_res:
            x_ref, a_ref, r_ref, wua_ref, wub_ref, b1_ref, w2_ref, b2_ref, y_ref = refs
        else:
            x_ref, a_ref, wua_ref, wub_ref, b1_ref, w2_ref, b2_ref, y_ref = refs
        h = (jnp.dot(x_ref[...], wua_ref[...], preferred_element_type=F32)
             + jnp.dot(a_ref[...], wub_ref[...], preferred_element_type=F32)
             + b1_ref[...])
        h = jnp.maximum(h, 0.0)
        u = jnp.dot(h, w2_ref[...], preferred_element_type=F32) + b2_ref[...]
        u = jnp.maximum(u, 0.0)
        if with_res:
            u = u + r_ref[...]
        y_ref[...] = u
    row_spec = pl.BlockSpec((T, H), lambda i: (i, 0))
    w_spec = pl.BlockSpec((H, H), lambda i: (0, 0))
    b_spec = pl.BlockSpec((1, H), lambda i: (0, 0))
    in_specs = [row_spec, row_spec] + ([row_spec] if with_res else []) + \
               [w_spec, w_spec, b_spec, w_spec, b_spec]
    args = [x, agg] + ([residual] if with_res else []) + [wua, wub, bu1, wu2, bu2]
    return pl.pallas_call(
        body,
        grid=(BN // T,),
        in_specs=in_specs,
        out_specs=row_spec,
        out_shape=jax.ShapeDtypeStruct((BN, H), F32),
    )(*args)


def _attention_readout(x3, scale, wd, bd):
    Bb, Nn, H = x3.shape
    OUT = wd.shape[1]
    TQ = 512
    NJ = Nn // TQ
    def body(xq_ref, xk_ref, scale_ref, wd_ref, bd_ref, o_ref, acc_ref):
        j = pl.program_id(1)
        xq = xq_ref[0]
        xk = xk_ref[0]
        s = lax.dot_general(xq, xk, (((1,), (1,)), ((), ())),
                            preferred_element_type=F32)
        s = s * scale_ref[0]
        mx = jnp.max(s, axis=1, keepdims=True)
        p = jnp.exp(s - mx)
        denom = jnp.sum(p, axis=1, keepdims=True)
        att = jnp.dot(p / denom, xk, preferred_element_type=F32)
        part = jnp.sum(att, axis=0, keepdims=True)
        @pl.when(j == 0)
        def _():
            acc_ref[...] = part
        @pl.when(j > 0)
        def _():
            acc_ref[...] = acc_ref[...] + part
        @pl.when(j == NJ - 1)
        def _():
            pooled = acc_ref[...] * (1.0 / Nn)
            o_ref[0] = (jnp.dot(pooled, wd_ref[...], preferred_element_type=F32)
                        + bd_ref[...])
    return pl.pallas_call(
        body,
        grid=(Bb, NJ),
        in_specs=[pl.BlockSpec((1, TQ, H), lambda b, j: (b, j, 0)),
                  pl.BlockSpec((1, Nn, H), lambda b, j: (b, 0, 0)),
                  pl.BlockSpec(memory_space=pltpu.SMEM),
                  pl.BlockSpec((H, OUT), lambda b, j: (0, 0)),
                  pl.BlockSpec((1, OUT), lambda b, j: (0, 0))],
        out_specs=pl.BlockSpec((1, 1, OUT), lambda b, j: (b, 0, 0)),
        out_shape=jax.ShapeDtypeStruct((Bb, 1, OUT), F32),
        scratch_shapes=[pltpu.VMEM((1, H), F32)],
    )(x3, x3, scale.reshape(1), wd, bd.reshape(1, OUT)).reshape(Bb, OUT)


@functools.lru_cache(maxsize=None)
def _make_gather_add(BN, BE, Nn, Ee, H):
    CH = 128
    EPW = BE // NW
    NCH = EPW // CH
    assert EPW * NW == BE and NCH * CH == EPW
    assert Ee % EPW == 0

    @functools.partial(
        pl.kernel, mesh=_sc_mesh(),
        out_type=jax.ShapeDtypeStruct((BE, H), F32),
        scratch_types=[pltpu.VMEM((CH,), jnp.int32),
                       pltpu.VMEM((CH,), jnp.int32),
                       pltpu.VMEM((CH, H), F32),
                       pltpu.VMEM((CH, H), F32),
                       pltpu.SemaphoreType.DMA,
                       pltpu.SemaphoreType.DMA])
    def gather_k(p_hbm, q_hbm, src_hbm, dst_hbm, out_hbm,
                 idx_s, idx_d, bufa, bufb, sema, semb):
        cid = lax.axis_index("c")
        sid = lax.axis_index("s")
        wid = sid * NC + cid
        base = wid * EPW
        boff = (base // Ee) * Nn

        @pl.loop(0, NCH)
        def _chunk(ch):
            ebase = pl.multiple_of(base + ch * CH, CH)
            pltpu.sync_copy(src_hbm.at[pl.ds(ebase, CH)], idx_s)
            pltpu.sync_copy(dst_hbm.at[pl.ds(ebase, CH)], idx_d)
            for i in range(CH // LANE):
                sl = pl.ds(i * LANE, LANE)
                idx_s[sl] = idx_s[sl] + boff
                idx_d[sl] = idx_d[sl] + boff
            cpa = pltpu.async_copy(p_hbm.at[idx_s], bufa, sema)
            cpb = pltpu.async_copy(q_hbm.at[idx_d], bufb, semb)
            cpa.wait()
            cpb.wait()

            @pl.loop(0, CH)
            def _row(r):
                for c in range(H // LANE):
                    sl = pl.ds(c * LANE, LANE)
                    bufa[r, sl] = bufa[r, sl] + bufb[r, sl]

            pltpu.sync_copy(bufa, out_hbm.at[pl.ds(ebase, CH)])

    return gather_k


@functools.lru_cache(maxsize=None)
def _make_scatter_add(BE, Bb, Nn, Ee, H):
    CH = 128
    EPT = Ee // NS
    NCH = EPT // CH
    BPC = Bb // NC
    ROWS_PT = Nn // NS
    assert NCH * CH == EPT and ROWS_PT * NS == Nn

    @functools.partial(
        pl.kernel, mesh=_sc_mesh(),
        out_type=jax.ShapeDtypeStruct((Bb * Nn, H), F32),
        scratch_types=[pltpu.VMEM((CH, H), F32),
                       pltpu.VMEM((NCH, CH), jnp.int32),
                       pltpu.VMEM((ROWS_PT, H), F32),
                       pltpu.VMEM_SHARED((Nn, H), F32)])
    def scatter_k(m_hbm, dst2d_hbm, out_hbm, mbuf, idxbuf, zbuf, shared):
        cid = lax.axis_index("c")
        sid = lax.axis_index("s")

        @pl.loop(0, ROWS_PT)
        def _z(r):
            for c in range(H // LANE):
                zbuf[r, pl.ds(c * LANE, LANE)] = jnp.zeros((LANE,), F32)

        for bi in range(BPC):
            b = cid * BPC + bi
            pltpu.sync_copy(zbuf, shared.at[pl.ds(sid * ROWS_PT, ROWS_PT)])
            plsc.subcore_barrier()
            ebase = pl.multiple_of(b * Ee + sid * EPT, CH)
            pltpu.sync_copy(dst2d_hbm.at[pl.ds(pl.multiple_of(ebase // CH, 8), NCH)],
                            idxbuf)

            @pl.loop(0, NCH)
            def _chunk(j):
                pltpu.sync_copy(
                    m_hbm.at[pl.ds(pl.multiple_of(ebase + j * CH, CH), CH)], mbuf)
                pltpu.sync_copy(mbuf, shared.at[idxbuf.at[j]], add=True)

            plsc.subcore_barrier()
            pltpu.sync_copy(
                shared.at[pl.ds(sid * ROWS_PT, ROWS_PT)],
                out_hbm.at[pl.ds(pl.multiple_of(b * Nn + sid * ROWS_PT, ROWS_PT), ROWS_PT)])
            plsc.subcore_barrier()

    return scatter_k


def kernel(node_features, edge_features, edge_src, edge_dst, params):
    B, N, D = node_features.shape
    _, E, DE = edge_features.shape
    H = params['layers'][0]['Wm2'].shape[0]
    BE = B * E

    x = node_features.reshape(B * N, D)
    ef = edge_features.reshape(BE, DE)
    src = edge_src.reshape(BE)
    dst = edge_dst.reshape(BE)
    dst2d = dst.reshape(BE // 128, 128)

    gather_k = _make_gather_add(B * N, BE, N, E, H)
    scatter_k = _make_scatter_add(BE, B, N, E, H)

    residual = None
    for p in params['layers']:
        W1 = p['Wm1']
        P, Q = _proj(x, W1[:H], W1[H:2 * H])
        g = gather_k(P, Q, src, dst)
        m = _edge_mlp(g, ef, W1[2 * H:], p['bm1'].reshape(1, H),
                      p['Wm2'], p['bm2'].reshape(1, H))
        agg = scatter_k(m, dst2d)
        x = _node_mlp(x, agg, p['Wu1'][:H], p['Wu1'][H:],
                      p['bu1'].reshape(1, H), p['Wu2'],
                      p['bu2'].reshape(1, H), residual)
        residual = x

    return _attention_readout(x.reshape(B, N, H), params['scale'],
                              params['Wd'], params['bd'])


# speedup vs baseline: 19.9706x; 1.3284x over previous
import functools

import jax
import jax.numpy as jnp
from jax import lax
from jax.experimental import pallas as pl
from jax.experimental.pallas import tpu as pltpu
from jax.experimental.pallas import tpu_sc as plsc

F32 = jnp.float32
NC, NS = 2, 16
NW = NC * NS
LANE = 16


def _sc_mesh():
    return plsc.VectorSubcoreMesh(core_axis_name="c", subcore_axis_name="s",
                                  num_cores=NC, num_subcores=NS)


def _proj(x, ws, wd):
    BN, D = x.shape
    H = ws.shape[1]
    T = 2048
    def body(x_ref, ws_ref, wd_ref, p_ref, q_ref):
        xv = x_ref[...]
        p_ref[...] = jnp.dot(xv, ws_ref[...], preferred_element_type=F32)
        q_ref[...] = jnp.dot(xv, wd_ref[...], preferred_element_type=F32)
    return pl.pallas_call(
        body,
        grid=(BN // T,),
        in_specs=[pl.BlockSpec((T, D), lambda i: (i, 0)),
                  pl.BlockSpec((D, H), lambda i: (0, 0)),
                  pl.BlockSpec((D, H), lambda i: (0, 0))],
        out_specs=[pl.BlockSpec((T, H), lambda i: (i, 0)),
                   pl.BlockSpec((T, H), lambda i: (i, 0))],
        out_shape=[jax.ShapeDtypeStruct((BN, H), F32),
                   jax.ShapeDtypeStruct((BN, H), F32)],
    )(x, ws, wd)


def _edge_mlp(g, ef, we, b1, w2, b2):
    BE, H = g.shape
    DE = ef.shape[1]
    T = 4096
    def body(g_ref, ef_ref, we_ref, b1_ref, w2_ref, b2_ref, m_ref):
        h = (g_ref[...]
             + jnp.dot(ef_ref[...], we_ref[...], preferred_element_type=F32)
             + b1_ref[...])
        h = jnp.maximum(h, 0.0)
        m = jnp.dot(h, w2_ref[...], preferred_element_type=F32) + b2_ref[...]
        m_ref[...] = jnp.maximum(m, 0.0)
    return pl.pallas_call(
        body,
        grid=(BE // T,),
        in_specs=[pl.BlockSpec((T, H), lambda i: (i, 0)),
                  pl.BlockSpec((T, DE), lambda i: (i, 0)),
                  pl.BlockSpec((DE, H), lambda i: (0, 0)),
                  pl.BlockSpec((1, H), lambda i: (0, 0)),
                  pl.BlockSpec((H, H), lambda i: (0, 0)),
                  pl.BlockSpec((1, H), lambda i: (0, 0))],
        out_specs=pl.BlockSpec((T, H), lambda i: (i, 0)),
        out_shape=jax.ShapeDtypeStruct((BE, H), F32),
    )(g, ef, we, b1, w2, b2)


def _node_mlp(x, agg, wua, wub, bu1, wu2, bu2, residual):
    BN, H = x.shape
    T = 2048
    with_res = residual is not None
    def body(*refs):
        if with_res:
            x_ref, a_ref, r_ref, wua_ref, wub_ref, b1_ref, w2_ref, b2_ref, y_ref = refs
        else:
            x_ref, a_ref, wua_ref, wub_ref, b1_ref, w2_ref, b2_ref, y_ref = refs
        h = (jnp.dot(x_ref[...], wua_ref[...], preferred_element_type=F32)
             + jnp.dot(a_ref[...], wub_ref[...], preferred_element_type=F32)
             + b1_ref[...])
        h = jnp.maximum(h, 0.0)
        u = jnp.dot(h, w2_ref[...], preferred_element_type=F32) + b2_ref[...]
        u = jnp.maximum(u, 0.0)
        if with_res:
            u = u + r_ref[...]
        y_ref[...] = u
    row_spec = pl.BlockSpec((T, H), lambda i: (i, 0))
    w_spec = pl.BlockSpec((H, H), lambda i: (0, 0))
    b_spec = pl.BlockSpec((1, H), lambda i: (0, 0))
    in_specs = [row_spec, row_spec] + ([row_spec] if with_res else []) + \
               [w_spec, w_spec, b_spec, w_spec, b_spec]
    args = [x, agg] + ([residual] if with_res else []) + [wua, wub, bu1, wu2, bu2]
    return pl.pallas_call(
        body,
        grid=(BN // T,),
        in_specs=in_specs,
        out_specs=row_spec,
        out_shape=jax.ShapeDtypeStruct((BN, H), F32),
    )(*args)


def _attention_readout(x3, scale, wd, bd):
    Bb, Nn, H = x3.shape
    OUT = wd.shape[1]
    TQ = 512
    NJ = Nn // TQ
    def body(xq_ref, xk_ref, scale_ref, wd_ref, bd_ref, o_ref, acc_ref):
        j = pl.program_id(1)
        xq = xq_ref[0]
        xk = xk_ref[0]
        s = lax.dot_general(xq, xk, (((1,), (1,)), ((), ())),
                            preferred_element_type=F32)
        s = s * scale_ref[0]
        mx = jnp.max(s, axis=1, keepdims=True)
        p = jnp.exp(s - mx)
        denom = jnp.sum(p, axis=1, keepdims=True)
        att = jnp.dot(p / denom, xk, preferred_element_type=F32)
        part = jnp.sum(att, axis=0, keepdims=True)
        @pl.when(j == 0)
        def _():
            acc_ref[...] = part
        @pl.when(j > 0)
        def _():
            acc_ref[...] = acc_ref[...] + part
        @pl.when(j == NJ - 1)
        def _():
            pooled = acc_ref[...] * (1.0 / Nn)
            o_ref[0] = (jnp.dot(pooled, wd_ref[...], preferred_element_type=F32)
                        + bd_ref[...])
    return pl.pallas_call(
        body,
        grid=(Bb, NJ),
        in_specs=[pl.BlockSpec((1, TQ, H), lambda b, j: (b, j, 0)),
                  pl.BlockSpec((1, Nn, H), lambda b, j: (b, 0, 0)),
                  pl.BlockSpec(memory_space=pltpu.SMEM),
                  pl.BlockSpec((H, OUT), lambda b, j: (0, 0)),
                  pl.BlockSpec((1, OUT), lambda b, j: (0, 0))],
        out_specs=pl.BlockSpec((1, 1, OUT), lambda b, j: (b, 0, 0)),
        out_shape=jax.ShapeDtypeStruct((Bb, 1, OUT), F32),
        scratch_shapes=[pltpu.VMEM((1, H), F32)],
    )(x3, x3, scale.reshape(1), wd, bd.reshape(1, OUT)).reshape(Bb, OUT)


@functools.lru_cache(maxsize=None)
def _make_gather_add(BN, BE, Nn, Ee, H):
    CH = 128
    EPW = BE // NW
    NCH = EPW // CH
    WCH = 16
    IB = WCH * CH
    NWIN = NCH // WCH
    assert EPW * NW == BE and NWIN * WCH == NCH
    assert Ee % EPW == 0

    @functools.partial(
        pl.kernel, mesh=_sc_mesh(),
        out_type=jax.ShapeDtypeStruct((BE, H), F32),
        scratch_types=[pltpu.VMEM((IB,), jnp.int32),
                       pltpu.VMEM((IB,), jnp.int32),
                       pltpu.VMEM((CH, H), F32), pltpu.VMEM((CH, H), F32),
                       pltpu.VMEM((CH, H), F32), pltpu.VMEM((CH, H), F32),
                       pltpu.VMEM((CH, H), F32), pltpu.VMEM((CH, H), F32),
                       pltpu.SemaphoreType.DMA, pltpu.SemaphoreType.DMA,
                       pltpu.SemaphoreType.DMA, pltpu.SemaphoreType.DMA,
                       pltpu.SemaphoreType.DMA, pltpu.SemaphoreType.DMA])
    def gather_k(p_hbm, q_hbm, src_hbm, dst_hbm, out_hbm,
                 idx_s, idx_d, bufa0, bufa1, bufb0, bufb1, wbuf0, wbuf1,
                 ga0, ga1, gb0, gb1, ws0, ws1):
        bufa, bufb, wbuf = (bufa0, bufa1), (bufb0, bufb1), (wbuf0, wbuf1)
        ga, gb, ws = (ga0, ga1), (gb0, gb1), (ws0, ws1)
        cid = lax.axis_index("c")
        sid = lax.axis_index("s")
        wid = sid * NC + cid
        base = wid * EPW
        boff = (base // Ee) * Nn

        @pl.loop(0, NWIN)
        def _win(w):
            wb = pl.multiple_of(base + w * IB, CH)
            pltpu.sync_copy(src_hbm.at[pl.ds(wb, IB)], idx_s)
            pltpu.sync_copy(dst_hbm.at[pl.ds(wb, IB)], idx_d)

            @pl.loop(0, IB // LANE)
            def _adj(r):
                sl = pl.ds(r * LANE, LANE)
                idx_s[sl] = idx_s[sl] + boff
                idx_d[sl] = idx_d[sl] + boff

            descs = {}

            def issue(c):
                s = c % 2
                descs[('a', s)] = pltpu.async_copy(
                    p_hbm.at[idx_s.at[pl.ds(c * CH, CH)]], bufa[s], ga[s])
                descs[('b', s)] = pltpu.async_copy(
                    q_hbm.at[idx_d.at[pl.ds(c * CH, CH)]], bufb[s], gb[s])

            issue(0)
            issue(1)
            for c in range(WCH):
                s = c % 2
                descs[('a', s)].wait()
                descs[('b', s)].wait()
                if c >= 2:
                    descs[('w', s)].wait()
                av, bv, wv = bufa[s], bufb[s], wbuf[s]

                @pl.loop(0, CH)
                def _row(r):
                    for k in range(H // LANE):
                        sl = pl.ds(k * LANE, LANE)
                        wv[r, sl] = av[r, sl] + bv[r, sl]

                descs[('w', s)] = pltpu.async_copy(
                    wv, out_hbm.at[pl.ds(pl.multiple_of(wb + c * CH, CH), CH)],
                    ws[s])
                if c + 2 < WCH:
                    issue(c + 2)
            descs[('w', 0)].wait()
            descs[('w', 1)].wait()

    return gather_k


@functools.lru_cache(maxsize=None)
def _make_scatter_add(BE, Bb, Nn, Ee, H):
    CH = 128
    EPT = Ee // NS
    NCH = EPT // CH
    BPC = Bb // NC
    ROWS_PT = Nn // NS
    assert NCH * CH == EPT and ROWS_PT * NS == Nn

    @functools.partial(
        pl.kernel, mesh=_sc_mesh(),
        out_type=jax.ShapeDtypeStruct((Bb * Nn, H), F32),
        scratch_types=[pltpu.VMEM((CH, H), F32),
                       pltpu.VMEM((NCH, CH), jnp.int32),
                       pltpu.VMEM((ROWS_PT, H), F32),
                       pltpu.VMEM_SHARED((Nn, H), F32)])
    def scatter_k(m_hbm, dst2d_hbm, out_hbm, mbuf, idxbuf, zbuf, shared):
        cid = lax.axis_index("c")
        sid = lax.axis_index("s")

        @pl.loop(0, ROWS_PT)
        def _z(r):
            for c in range(H // LANE):
                zbuf[r, pl.ds(c * LANE, LANE)] = jnp.zeros((LANE,), F32)

        for bi in range(BPC):
            b = cid * BPC + bi
            pltpu.sync_copy(zbuf, shared.at[pl.ds(sid * ROWS_PT, ROWS_PT)])
            plsc.subcore_barrier()
            ebase = pl.multiple_of(b * Ee + sid * EPT, CH)
            pltpu.sync_copy(dst2d_hbm.at[pl.ds(pl.multiple_of(ebase // CH, 8), NCH)],
                            idxbuf)

            @pl.loop(0, NCH)
            def _chunk(j):
                pltpu.sync_copy(
                    m_hbm.at[pl.ds(pl.multiple_of(ebase + j * CH, CH), CH)], mbuf)
                pltpu.sync_copy(mbuf, shared.at[idxbuf.at[j]], add=True)

            plsc.subcore_barrier()
            pltpu.sync_copy(
                shared.at[pl.ds(sid * ROWS_PT, ROWS_PT)],
                out_hbm.at[pl.ds(pl.multiple_of(b * Nn + sid * ROWS_PT, ROWS_PT), ROWS_PT)])
            plsc.subcore_barrier()

    return scatter_k


def kernel(node_features, edge_features, edge_src, edge_dst, params):
    B, N, D = node_features.shape
    _, E, DE = edge_features.shape
    H = params['layers'][0]['Wm2'].shape[0]
    BE = B * E

    x = node_features.reshape(B * N, D)
    ef = edge_features.reshape(BE, DE)
    src = edge_src.reshape(BE)
    dst = edge_dst.reshape(BE)
    dst2d = dst.reshape(BE // 128, 128)

    gather_k = _make_gather_add(B * N, BE, N, E, H)
    scatter_k = _make_scatter_add(BE, B, N, E, H)

    residual = None
    for p in params['layers']:
        W1 = p['Wm1']
        P, Q = _proj(x, W1[:H], W1[H:2 * H])
        g = gather_k(P, Q, src, dst)
        m = _edge_mlp(g, ef, W1[2 * H:], p['bm1'].reshape(1, H),
                      p['Wm2'], p['bm2'].reshape(1, H))
        agg = scatter_k(m, dst2d)
        x = _node_mlp(x, agg, p['Wu1'][:H], p['Wu1'][H:],
                      p['bu1'].reshape(1, H), p['Wu2'],
                      p['bu2'].reshape(1, H), residual)
        residual = x

    return _attention_readout(x.reshape(B, N, H), params['scale'],
                              params['Wd'], params['bd'])


# speedup vs baseline: 22.3903x; 1.1212x over previous
import functools

import jax
import jax.numpy as jnp
from jax import lax
from jax.experimental import pallas as pl
from jax.experimental.pallas import tpu as pltpu
from jax.experimental.pallas import tpu_sc as plsc

F32 = jnp.float32
NC, NS = 2, 16
NW = NC * NS
LANE = 16


def _sc_mesh():
    return plsc.VectorSubcoreMesh(core_axis_name="c", subcore_axis_name="s",
                                  num_cores=NC, num_subcores=NS)


def _proj(x, ws, wd):
    BN, D = x.shape
    H = ws.shape[1]
    T = 2048
    def body(x_ref, ws_ref, wd_ref, p_ref, q_ref):
        xv = x_ref[...]
        p_ref[...] = jnp.dot(xv, ws_ref[...], preferred_element_type=F32)
        q_ref[...] = jnp.dot(xv, wd_ref[...], preferred_element_type=F32)
    return pl.pallas_call(
        body,
        grid=(BN // T,),
        in_specs=[pl.BlockSpec((T, D), lambda i: (i, 0)),
                  pl.BlockSpec((D, H), lambda i: (0, 0)),
                  pl.BlockSpec((D, H), lambda i: (0, 0))],
        out_specs=[pl.BlockSpec((T, H), lambda i: (i, 0)),
                   pl.BlockSpec((T, H), lambda i: (i, 0))],
        out_shape=[jax.ShapeDtypeStruct((BN, H), F32),
                   jax.ShapeDtypeStruct((BN, H), F32)],
    )(x, ws, wd)


def _edge_mlp(g, ef, we, b1, w2, b2):
    BE, H = g.shape
    DE = ef.shape[1]
    T = 4096
    def body(g_ref, ef_ref, we_ref, b1_ref, w2_ref, b2_ref, m_ref):
        h = (g_ref[...]
             + jnp.dot(ef_ref[...], we_ref[...], preferred_element_type=F32)
             + b1_ref[...])
        h = jnp.maximum(h, 0.0)
        m = jnp.dot(h, w2_ref[...], preferred_element_type=F32) + b2_ref[...]
        m_ref[...] = jnp.maximum(m, 0.0)
    return pl.pallas_call(
        body,
        grid=(BE // T,),
        in_specs=[pl.BlockSpec((T, H), lambda i: (i, 0)),
                  pl.BlockSpec((T, DE), lambda i: (i, 0)),
                  pl.BlockSpec((DE, H), lambda i: (0, 0)),
                  pl.BlockSpec((1, H), lambda i: (0, 0)),
                  pl.BlockSpec((H, H), lambda i: (0, 0)),
                  pl.BlockSpec((1, H), lambda i: (0, 0))],
        out_specs=pl.BlockSpec((T, H), lambda i: (i, 0)),
        out_shape=jax.ShapeDtypeStruct((BE, H), F32),
    )(g, ef, we, b1, w2, b2)


def _node_mlp(x, agg, wua, wub, bu1, wu2, bu2, residual):
    BN, H = x.shape
    T = 2048
    with_res = residual is not None
    def body(*refs):
        if with_res:
            x_ref, a_ref, r_ref, wua_ref, wub_ref, b1_ref, w2_ref, b2_ref, y_ref = refs
        else:
            x_ref, a_ref, wua_ref, wub_ref, b1_ref, w2_ref, b2_ref, y_ref = refs
        h = (jnp.dot(x_ref[...], wua_ref[...], preferred_element_type=F32)
             + jnp.dot(a_ref[...], wub_ref[...], preferred_element_type=F32)
             + b1_ref[...])
        h = jnp.maximum(h, 0.0)
        u = jnp.dot(h, w2_ref[...], preferred_element_type=F32) + b2_ref[...]
        u = jnp.maximum(u, 0.0)
        if with_res:
            u = u + r_ref[...]
        y_ref[...] = u
    row_spec = pl.BlockSpec((T, H), lambda i: (i, 0))
    w_spec = pl.BlockSpec((H, H), lambda i: (0, 0))
    b_spec = pl.BlockSpec((1, H), lambda i: (0, 0))
    in_specs = [row_spec, row_spec] + ([row_spec] if with_res else []) + \
               [w_spec, w_spec, b_spec, w_spec, b_spec]
    args = [x, agg] + ([residual] if with_res else []) + [wua, wub, bu1, wu2, bu2]
    return pl.pallas_call(
        body,
        grid=(BN // T,),
        in_specs=in_specs,
        out_specs=row_spec,
        out_shape=jax.ShapeDtypeStruct((BN, H), F32),
    )(*args)


def _attention_readout(x3, scale, wd, bd):
    Bb, Nn, H = x3.shape
    OUT = wd.shape[1]
    TQ = 512
    NJ = Nn // TQ
    def body(xq_ref, xk_ref, scale_ref, wd_ref, bd_ref, o_ref, acc_ref):
        j = pl.program_id(1)
        xq = xq_ref[0]
        xk = xk_ref[0]
        s = lax.dot_general(xq, xk, (((1,), (1,)), ((), ())),
                            preferred_element_type=F32)
        s = s * scale_ref[0]
        mx = jnp.max(s, axis=1, keepdims=True)
        p = jnp.exp(s - mx)
        denom = jnp.sum(p, axis=1, keepdims=True)
        att = jnp.dot(p / denom, xk, preferred_element_type=F32)
        part = jnp.sum(att, axis=0, keepdims=True)
        @pl.when(j == 0)
        def _():
            acc_ref[...] = part
        @pl.when(j > 0)
        def _():
            acc_ref[...] = acc_ref[...] + part
        @pl.when(j == NJ - 1)
        def _():
            pooled = acc_ref[...] * (1.0 / Nn)
            o_ref[0] = (jnp.dot(pooled, wd_ref[...], preferred_element_type=F32)
                        + bd_ref[...])
    return pl.pallas_call(
        body,
        grid=(Bb, NJ),
        in_specs=[pl.BlockSpec((1, TQ, H), lambda b, j: (b, j, 0)),
                  pl.BlockSpec((1, Nn, H), lambda b, j: (b, 0, 0)),
                  pl.BlockSpec(memory_space=pltpu.SMEM),
                  pl.BlockSpec((H, OUT), lambda b, j: (0, 0)),
                  pl.BlockSpec((1, OUT), lambda b, j: (0, 0))],
        out_specs=pl.BlockSpec((1, 1, OUT), lambda b, j: (b, 0, 0)),
        out_shape=jax.ShapeDtypeStruct((Bb, 1, OUT), F32),
        scratch_shapes=[pltpu.VMEM((1, H), F32)],
    )(x3, x3, scale.reshape(1), wd, bd.reshape(1, OUT)).reshape(Bb, OUT)


@functools.lru_cache(maxsize=None)
def _make_gather_add(BN, BE, Nn, Ee, H):
    CH = 128
    EPW = BE // NW
    NCH = EPW // CH
    WCH = 16
    IB = WCH * CH
    NWIN = NCH // WCH
    assert EPW * NW == BE and NWIN * WCH == NCH
    assert Ee % EPW == 0

    @functools.partial(
        pl.kernel, mesh=_sc_mesh(),
        out_type=jax.ShapeDtypeStruct((BE, H), F32),
        scratch_types=[pltpu.VMEM((IB,), jnp.int32),
                       pltpu.VMEM((IB,), jnp.int32),
                       pltpu.VMEM((CH, H), F32), pltpu.VMEM((CH, H), F32),
                       pltpu.VMEM((CH, H), F32), pltpu.VMEM((CH, H), F32),
                       pltpu.VMEM((CH, H), F32), pltpu.VMEM((CH, H), F32),
                       pltpu.SemaphoreType.DMA, pltpu.SemaphoreType.DMA,
                       pltpu.SemaphoreType.DMA, pltpu.SemaphoreType.DMA,
                       pltpu.SemaphoreType.DMA, pltpu.SemaphoreType.DMA])
    def gather_k(p_hbm, q_hbm, src_hbm, dst_hbm, out_hbm,
                 idx_s, idx_d, bufa0, bufa1, bufb0, bufb1, wbuf0, wbuf1,
                 ga0, ga1, gb0, gb1, ws0, ws1):
        bufa, bufb, wbuf = (bufa0, bufa1), (bufb0, bufb1), (wbuf0, wbuf1)
        ga, gb, ws = (ga0, ga1), (gb0, gb1), (ws0, ws1)
        cid = lax.axis_index("c")
        sid = lax.axis_index("s")
        wid = sid * NC + cid
        base = wid * EPW
        boff = (base // Ee) * Nn

        @pl.loop(0, NWIN)
        def _win(w):
            wb = pl.multiple_of(base + w * IB, CH)
            pltpu.sync_copy(src_hbm.at[pl.ds(wb, IB)], idx_s)
            pltpu.sync_copy(dst_hbm.at[pl.ds(wb, IB)], idx_d)

            @pl.loop(0, IB // LANE)
            def _adj(r):
                sl = pl.ds(r * LANE, LANE)
                idx_s[sl] = idx_s[sl] + boff
                idx_d[sl] = idx_d[sl] + boff

            descs = {}

            def issue(c):
                s = c % 2
                descs[('a', s)] = pltpu.async_copy(
                    p_hbm.at[idx_s.at[pl.ds(c * CH, CH)]], bufa[s], ga[s])
                descs[('b', s)] = pltpu.async_copy(
                    q_hbm.at[idx_d.at[pl.ds(c * CH, CH)]], bufb[s], gb[s])

            issue(0)
            issue(1)
            for c in range(WCH):
                s = c % 2
                descs[('a', s)].wait()
                descs[('b', s)].wait()
                if c >= 2:
                    descs[('w', s)].wait()
                av, bv, wv = bufa[s], bufb[s], wbuf[s]

                @pl.loop(0, CH)
                def _row(r):
                    for k in range(H // LANE):
                        sl = pl.ds(k * LANE, LANE)
                        wv[r, sl] = av[r, sl] + bv[r, sl]

                descs[('w', s)] = pltpu.async_copy(
                    wv, out_hbm.at[pl.ds(pl.multiple_of(wb + c * CH, CH), CH)],
                    ws[s])
                if c + 2 < WCH:
                    issue(c + 2)
            descs[('w', 0)].wait()
            descs[('w', 1)].wait()

    return gather_k


@functools.lru_cache(maxsize=None)
def _make_scatter_add(BE, Bb, Nn, Ee, H):
    CH = 128
    EPT = Ee // NS
    NCH = EPT // CH
    BPC = Bb // NC
    ROWS_PT = Nn // NS
    assert NCH * CH == EPT and ROWS_PT * NS == Nn

    NBUF = 4

    @functools.partial(
        pl.kernel, mesh=_sc_mesh(),
        out_type=jax.ShapeDtypeStruct((Bb * Nn, H), F32),
        scratch_types=[pltpu.VMEM((CH, H), F32), pltpu.VMEM((CH, H), F32),
                       pltpu.VMEM((CH, H), F32), pltpu.VMEM((CH, H), F32),
                       pltpu.VMEM((NCH, CH), jnp.int32),
                       pltpu.VMEM((ROWS_PT, H), F32),
                       pltpu.VMEM_SHARED((Nn, H), F32),
                       pltpu.SemaphoreType.DMA, pltpu.SemaphoreType.DMA,
                       pltpu.SemaphoreType.DMA, pltpu.SemaphoreType.DMA,
                       pltpu.SemaphoreType.DMA, pltpu.SemaphoreType.DMA,
                       pltpu.SemaphoreType.DMA, pltpu.SemaphoreType.DMA])
    def scatter_k(m_hbm, dst2d_hbm, out_hbm, mbuf0, mbuf1, mbuf2, mbuf3,
                  idxbuf, zbuf, shared, l0, l1, l2, l3, s0, s1, s2, s3):
        mbuf = (mbuf0, mbuf1, mbuf2, mbuf3)
        lsem = (l0, l1, l2, l3)
        ssem = (s0, s1, s2, s3)
        cid = lax.axis_index("c")
        sid = lax.axis_index("s")

        @pl.loop(0, ROWS_PT)
        def _z(r):
            for c in range(H // LANE):
                zbuf[r, pl.ds(c * LANE, LANE)] = jnp.zeros((LANE,), F32)

        @pl.loop(0, BPC)
        def _batch(bi):
            b = cid * BPC + bi
            pltpu.sync_copy(zbuf, shared.at[pl.ds(sid * ROWS_PT, ROWS_PT)])
            plsc.subcore_barrier()
            ebase = pl.multiple_of(b * Ee + sid * EPT, CH)
            pltpu.sync_copy(dst2d_hbm.at[pl.ds(pl.multiple_of(ebase // CH, 8), NCH)],
                            idxbuf)

            descs = {}

            def load(j):
                s = j % NBUF
                descs[('l', s)] = pltpu.async_copy(
                    m_hbm.at[pl.ds(pl.multiple_of(ebase + j * CH, CH), CH)],
                    mbuf[s], lsem[s])

            load(0)
            load(1)
            for j in range(NCH):
                s = j % NBUF
                descs[('l', s)].wait()
                descs[('s', s)] = pltpu.async_copy(
                    mbuf[s], shared.at[idxbuf.at[j]], ssem[s], add=True)
                pj = j + 2
                if pj < NCH:
                    ps = pj % NBUF
                    if pj >= NBUF:
                        descs[('s', ps)].wait()
                    load(pj)
            for j in range(NCH - NBUF, NCH):
                descs[('s', j % NBUF)].wait()

            plsc.subcore_barrier()
            pltpu.sync_copy(
                shared.at[pl.ds(sid * ROWS_PT, ROWS_PT)],
                out_hbm.at[pl.ds(pl.multiple_of(b * Nn + sid * ROWS_PT, ROWS_PT),
                                 ROWS_PT)])
            plsc.subcore_barrier()

    return scatter_k


def kernel(node_features, edge_features, edge_src, edge_dst, params):
    B, N, D = node_features.shape
    _, E, DE = edge_features.shape
    H = params['layers'][0]['Wm2'].shape[0]
    BE = B * E

    x = node_features.reshape(B * N, D)
    ef = edge_features.reshape(BE, DE)
    src = edge_src.reshape(BE)
    dst = edge_dst.reshape(BE)
    dst2d = dst.reshape(BE // 128, 128)

    gather_k = _make_gather_add(B * N, BE, N, E, H)
    scatter_k = _make_scatter_add(BE, B, N, E, H)

    residual = None
    for p in params['layers']:
        W1 = p['Wm1']
        P, Q = _proj(x, W1[:H], W1[H:2 * H])
        g = gather_k(P, Q, src, dst)
        m = _edge_mlp(g, ef, W1[2 * H:], p['bm1'].reshape(1, H),
                      p['Wm2'], p['bm2'].reshape(1, H))
        agg = scatter_k(m, dst2d)
        x = _node_mlp(x, agg, p['Wu1'][:H], p['Wu1'][H:],
                      p['bu1'].reshape(1, H), p['Wu2'],
                      p['bu2'].reshape(1, H), residual)
        residual = x

    return _attention_readout(x.reshape(B, N, H), params['scale'],
                              params['Wd'], params['bd'])


# speedup vs baseline: 23.1705x; 1.0348x over previous
import functools

import jax
import jax.numpy as jnp
from jax import lax
from jax.experimental import pallas as pl
from jax.experimental.pallas import tpu as pltpu
from jax.experimental.pallas import tpu_sc as plsc

F32 = jnp.float32
NC, NS = 2, 16
NW = NC * NS
LANE = 16


def _sc_mesh():
    return plsc.VectorSubcoreMesh(core_axis_name="c", subcore_axis_name="s",
                                  num_cores=NC, num_subcores=NS)


def _proj(x, ws, wd):
    BN, D = x.shape
    H = ws.shape[1]
    T = 2048
    def body(x_ref, ws_ref, wd_ref, p_ref, q_ref):
        xv = x_ref[...]
        p_ref[...] = jnp.dot(xv, ws_ref[...], preferred_element_type=F32)
        q_ref[...] = jnp.dot(xv, wd_ref[...], preferred_element_type=F32)
    return pl.pallas_call(
        body,
        grid=(BN // T,),
        in_specs=[pl.BlockSpec((T, D), lambda i: (i, 0)),
                  pl.BlockSpec((D, H), lambda i: (0, 0)),
                  pl.BlockSpec((D, H), lambda i: (0, 0))],
        out_specs=[pl.BlockSpec((T, H), lambda i: (i, 0)),
                   pl.BlockSpec((T, H), lambda i: (i, 0))],
        out_shape=[jax.ShapeDtypeStruct((BN, H), F32),
                   jax.ShapeDtypeStruct((BN, H), F32)],
    )(x, ws, wd)


def _edge_mlp(g, ef, we, b1, w2, b2):
    BE, H = g.shape
    DE = ef.shape[1]
    T = 4096
    def body(g_ref, ef_ref, we_ref, b1_ref, w2_ref, b2_ref, m_ref):
        h = (g_ref[...]
             + jnp.dot(ef_ref[...], we_ref[...], preferred_element_type=F32)
             + b1_ref[...])
        h = jnp.maximum(h, 0.0)
        m = jnp.dot(h, w2_ref[...], preferred_element_type=F32) + b2_ref[...]
        m_ref[...] = jnp.maximum(m, 0.0)
    return pl.pallas_call(
        body,
        grid=(BE // T,),
        in_specs=[pl.BlockSpec((T, H), lambda i: (i, 0)),
                  pl.BlockSpec((T, DE), lambda i: (i, 0)),
                  pl.BlockSpec((DE, H), lambda i: (0, 0)),
                  pl.BlockSpec((1, H), lambda i: (0, 0)),
                  pl.BlockSpec((H, H), lambda i: (0, 0)),
                  pl.BlockSpec((1, H), lambda i: (0, 0))],
        out_specs=pl.BlockSpec((T, H), lambda i: (i, 0)),
        out_shape=jax.ShapeDtypeStruct((BE, H), F32),
    )(g, ef, we, b1, w2, b2)


def _node_mlp(x, agg, wua, wub, bu1, wu2, bu2, residual):
    BN, H = x.shape
    T = 2048
    with_res = residual is not None
    def body(*refs):
        if with_res:
            x_ref, a_ref, r_ref, wua_ref, wub_ref, b1_ref, w2_ref, b2_ref, y_ref = refs
        else:
            x_ref, a_ref, wua_ref, wub_ref, b1_ref, w2_ref, b2_ref, y_ref = refs
        h = (jnp.dot(x_ref[...], wua_ref[...], preferred_element_type=F32)
             + jnp.dot(a_ref[...], wub_ref[...], preferred_element_type=F32)
             + b1_ref[...])
        h = jnp.maximum(h, 0.0)
        u = jnp.dot(h, w2_ref[...], preferred_element_type=F32) + b2_ref[...]
        u = jnp.maximum(u, 0.0)
        if with_res:
            u = u + r_ref[...]
        y_ref[...] = u
    row_spec = pl.BlockSpec((T, H), lambda i: (i, 0))
    w_spec = pl.BlockSpec((H, H), lambda i: (0, 0))
    b_spec = pl.BlockSpec((1, H), lambda i: (0, 0))
    in_specs = [row_spec, row_spec] + ([row_spec] if with_res else []) + \
               [w_spec, w_spec, b_spec, w_spec, b_spec]
    args = [x, agg] + ([residual] if with_res else []) + [wua, wub, bu1, wu2, bu2]
    return pl.pallas_call(
        body,
        grid=(BN // T,),
        in_specs=in_specs,
        out_specs=row_spec,
        out_shape=jax.ShapeDtypeStruct((BN, H), F32),
    )(*args)


def _attention_readout(x3, scale, wd, bd):
    Bb, Nn, H = x3.shape
    OUT = wd.shape[1]
    TQ = 512
    NJ = Nn // TQ
    def body(xq_ref, xk_ref, scale_ref, wd_ref, bd_ref, o_ref, acc_ref):
        j = pl.program_id(1)
        xq = xq_ref[0]
        xk = xk_ref[0]
        s = lax.dot_general(xq, xk, (((1,), (1,)), ((), ())),
                            preferred_element_type=F32)
        s = s * scale_ref[0]
        mx = jnp.max(s, axis=1, keepdims=True)
        p = jnp.exp(s - mx)
        denom = jnp.sum(p, axis=1, keepdims=True)
        att = jnp.dot(p / denom, xk, preferred_element_type=F32)
        part = jnp.sum(att, axis=0, keepdims=True)
        @pl.when(j == 0)
        def _():
            acc_ref[...] = part
        @pl.when(j > 0)
        def _():
            acc_ref[...] = acc_ref[...] + part
        @pl.when(j == NJ - 1)
        def _():
            pooled = acc_ref[...] * (1.0 / Nn)
            o_ref[0] = (jnp.dot(pooled, wd_ref[...], preferred_element_type=F32)
                        + bd_ref[...])
    return pl.pallas_call(
        body,
        grid=(Bb, NJ),
        in_specs=[pl.BlockSpec((1, TQ, H), lambda b, j: (b, j, 0)),
                  pl.BlockSpec((1, Nn, H), lambda b, j: (b, 0, 0)),
                  pl.BlockSpec(memory_space=pltpu.SMEM),
                  pl.BlockSpec((H, OUT), lambda b, j: (0, 0)),
                  pl.BlockSpec((1, OUT), lambda b, j: (0, 0))],
        out_specs=pl.BlockSpec((1, 1, OUT), lambda b, j: (b, 0, 0)),
        out_shape=jax.ShapeDtypeStruct((Bb, 1, OUT), F32),
        scratch_shapes=[pltpu.VMEM((1, H), F32)],
    )(x3, x3, scale.reshape(1), wd, bd.reshape(1, OUT)).reshape(Bb, OUT)


@functools.lru_cache(maxsize=None)
def _make_gather_add(BN, BE, Nn, Ee, H):
    CH = 128
    EPW = BE // NW
    NCH = EPW // CH
    WCH = 16
    IB = WCH * CH
    NWIN = NCH // WCH
    assert EPW * NW == BE and NWIN * WCH == NCH
    assert Ee % EPW == 0

    @functools.partial(
        pl.kernel, mesh=_sc_mesh(),
        out_type=jax.ShapeDtypeStruct((BE, H), F32),
        scratch_types=[pltpu.VMEM((IB,), jnp.int32),
                       pltpu.VMEM((IB,), jnp.int32),
                       pltpu.VMEM((CH, H), F32), pltpu.VMEM((CH, H), F32),
                       pltpu.VMEM((CH, H), F32), pltpu.VMEM((CH, H), F32),
                       pltpu.VMEM((CH, H), F32), pltpu.VMEM((CH, H), F32),
                       pltpu.SemaphoreType.DMA, pltpu.SemaphoreType.DMA,
                       pltpu.SemaphoreType.DMA, pltpu.SemaphoreType.DMA,
                       pltpu.SemaphoreType.DMA, pltpu.SemaphoreType.DMA])
    def gather_k(p_hbm, q_hbm, src_hbm, dst_hbm, out_hbm,
                 idx_s, idx_d, bufa0, bufa1, bufb0, bufb1, wbuf0, wbuf1,
                 ga0, ga1, gb0, gb1, ws0, ws1):
        bufa, bufb, wbuf = (bufa0, bufa1), (bufb0, bufb1), (wbuf0, wbuf1)
        ga, gb, ws = (ga0, ga1), (gb0, gb1), (ws0, ws1)
        cid = lax.axis_index("c")
        sid = lax.axis_index("s")
        wid = sid * NC + cid
        base = wid * EPW
        boff = (base // Ee) * Nn

        @pl.loop(0, NWIN)
        def _win(w):
            wb = pl.multiple_of(base + w * IB, CH)
            pltpu.sync_copy(src_hbm.at[pl.ds(wb, IB)], idx_s)
            pltpu.sync_copy(dst_hbm.at[pl.ds(wb, IB)], idx_d)

            @pl.loop(0, IB // LANE)
            def _adj(r):
                sl = pl.ds(r * LANE, LANE)
                idx_s[sl] = idx_s[sl] + boff
                idx_d[sl] = idx_d[sl] + boff

            descs = {}

            def issue(c):
                s = c % 2
                descs[('a', s)] = pltpu.async_copy(
                    p_hbm.at[idx_s.at[pl.ds(c * CH, CH)]], bufa[s], ga[s])
                descs[('b', s)] = pltpu.async_copy(
                    q_hbm.at[idx_d.at[pl.ds(c * CH, CH)]], bufb[s], gb[s])

            issue(0)
            issue(1)
            for c in range(WCH):
                s = c % 2
                descs[('a', s)].wait()
                descs[('b', s)].wait()
                if c >= 2:
                    descs[('w', s)].wait()
                av, bv, wv = bufa[s], bufb[s], wbuf[s]

                @pl.loop(0, CH)
                def _row(r):
                    for k in range(H // LANE):
                        sl = pl.ds(k * LANE, LANE)
                        wv[r, sl] = av[r, sl] + bv[r, sl]

                descs[('w', s)] = pltpu.async_copy(
                    wv, out_hbm.at[pl.ds(pl.multiple_of(wb + c * CH, CH), CH)],
                    ws[s])
                if c + 2 < WCH:
                    issue(c + 2)
            descs[('w', 0)].wait()
            descs[('w', 1)].wait()

    return gather_k


@functools.lru_cache(maxsize=None)
def _make_scatter_add(BE, Bb, Nn, Ee, H):
    CH = 128
    EPT = Ee // NS
    NCH = EPT // CH
    BPC = Bb // NC
    ROWS_PT = Nn // NS
    assert NCH * CH == EPT and ROWS_PT * NS == Nn

    NBUF = 4

    @functools.partial(
        pl.kernel, mesh=_sc_mesh(),
        out_type=jax.ShapeDtypeStruct((Bb * Nn, H), F32),
        scratch_types=[pltpu.VMEM((CH, H), F32), pltpu.VMEM((CH, H), F32),
                       pltpu.VMEM((CH, H), F32), pltpu.VMEM((CH, H), F32),
                       pltpu.VMEM((NCH, CH), jnp.int32),
                       pltpu.VMEM((ROWS_PT, H), F32),
                       pltpu.VMEM_SHARED((Nn, H), F32),
                       pltpu.SemaphoreType.DMA, pltpu.SemaphoreType.DMA,
                       pltpu.SemaphoreType.DMA, pltpu.SemaphoreType.DMA,
                       pltpu.SemaphoreType.DMA, pltpu.SemaphoreType.DMA,
                       pltpu.SemaphoreType.DMA, pltpu.SemaphoreType.DMA])
    def scatter_k(m_hbm, dst2d_hbm, out_hbm, mbuf0, mbuf1, mbuf2, mbuf3,
                  idxbuf, zbuf, shared, l0, l1, l2, l3, s0, s1, s2, s3):
        mbuf = (mbuf0, mbuf1, mbuf2, mbuf3)
        lsem = (l0, l1, l2, l3)
        ssem = (s0, s1, s2, s3)
        cid = lax.axis_index("c")
        sid = lax.axis_index("s")

        @pl.loop(0, ROWS_PT)
        def _z(r):
            for c in range(H // LANE):
                zbuf[r, pl.ds(c * LANE, LANE)] = jnp.zeros((LANE,), F32)

        @pl.loop(0, BPC)
        def _batch(bi):
            b = cid * BPC + bi
            pltpu.sync_copy(zbuf, shared.at[pl.ds(sid * ROWS_PT, ROWS_PT)])
            plsc.subcore_barrier()
            ebase = pl.multiple_of(b * Ee + sid * EPT, CH)
            pltpu.sync_copy(dst2d_hbm.at[pl.ds(pl.multiple_of(ebase // CH, 8), NCH)],
                            idxbuf)

            descs = {}

            def load(j):
                s = j % NBUF
                descs[('l', s)] = pltpu.async_copy(
                    m_hbm.at[pl.ds(pl.multiple_of(ebase + j * CH, CH), CH)],
                    mbuf[s], lsem[s])

            load(0)
            load(1)
            for j in range(NCH):
                s = j % NBUF
                descs[('l', s)].wait()
                descs[('s', s)] = pltpu.async_copy(
                    mbuf[s], shared.at[idxbuf.at[j]], ssem[s], add=True)
                pj = j + 2
                if pj < NCH:
                    ps = pj % NBUF
                    if pj >= NBUF:
                        descs[('s', ps)].wait()
                    load(pj)
            for j in range(NCH - NBUF, NCH):
                descs[('s', j % NBUF)].wait()

            plsc.subcore_barrier()
            pltpu.sync_copy(
                shared.at[pl.ds(sid * ROWS_PT, ROWS_PT)],
                out_hbm.at[pl.ds(pl.multiple_of(b * Nn + sid * ROWS_PT, ROWS_PT),
                                 ROWS_PT)])
            plsc.subcore_barrier()

    return scatter_k


def kernel(node_features, edge_features, edge_src, edge_dst, params):
    B, N, D = node_features.shape
    _, E, DE = edge_features.shape
    H = params['layers'][0]['Wm2'].shape[0]
    BE = B * E

    Bh = B // 2
    BEh = Bh * E
    gather_k = _make_gather_add(Bh * N, BEh, N, E, H)
    scatter_k = _make_scatter_add(BEh, Bh, N, E, H)

    halves = []
    for hh in range(2):
        sl = slice(hh * Bh, (hh + 1) * Bh)
        x = node_features[sl].reshape(Bh * N, D)
        ef = edge_features[sl].reshape(BEh, DE)
        src = edge_src[sl].reshape(BEh)
        dst = edge_dst[sl].reshape(BEh)
        dst2d = dst.reshape(BEh // 128, 128)

        residual = None
        for p in params['layers']:
            W1 = p['Wm1']
            P, Q = _proj(x, W1[:H], W1[H:2 * H])
            g = gather_k(P, Q, src, dst)
            m = _edge_mlp(g, ef, W1[2 * H:], p['bm1'].reshape(1, H),
                          p['Wm2'], p['bm2'].reshape(1, H))
            agg = scatter_k(m, dst2d)
            x = _node_mlp(x, agg, p['Wu1'][:H], p['Wu1'][H:],
                          p['bu1'].reshape(1, H), p['Wu2'],
                          p['bu2'].reshape(1, H), residual)
            residual = x
        halves.append(x.reshape(Bh, N, H))

    xfull = jnp.concatenate(halves, axis=0)
    return _attention_readout(xfull, params['scale'],
                              params['Wd'], params['bd'])
